# baseline (speedup 1.0000x reference)
import functools

import jax
import jax.numpy as jnp
from jax import lax
from jax.experimental import pallas as pl
from jax.experimental.pallas import tpu as pltpu
from jax.experimental.pallas import tpu_sc as plsc

N = 10000
E = 320000
D_IN = 128
HID = 64
OUT = 112

NC = 2
NS = 16
NW = NC * NS
EPT = E // NW
CH = 80
NCH = EPT // CH
RA = 624
RTAIL = N - NS * RA
W1 = 128
W2 = 128


def _make_agg(W):
  mesh = plsc.VectorSubcoreMesh(core_axis_name="c", subcore_axis_name="s")

  @functools.partial(
      pl.kernel,
      out_type=jax.ShapeDtypeStruct((NC, N, W), jnp.float32),
      mesh=mesh,
      scratch_types=[
          pltpu.VMEM((EPT,), jnp.int32),
          pltpu.VMEM((NCH, CH), jnp.int32),
          pltpu.VMEM((CH, W), jnp.float32),
          pltpu.VMEM((CH, W), jnp.float32),
          pltpu.VMEM_SHARED((N, W), jnp.float32),
          pltpu.SemaphoreType.DMA,
          pltpu.SemaphoreType.DMA,
      ],
  )
  def agg(y_hbm, src_hbm, dst_hbm, out_hbm, src_v, dst_v, rows_a, rows_b,
          acc_sh, sem_a, sem_b):
    c = lax.axis_index("c")
    s = lax.axis_index("s")
    wid = s * NC + c

    @pl.loop(0, CH)
    def _(i):
      @pl.loop(0, W // 16)
      def _(k):
        rows_a[i, pl.ds(k * 16, 16)] = jnp.zeros((16,), jnp.float32)

    @pl.loop(0, RA // CH)
    def _(i):
      pltpu.sync_copy(rows_a, acc_sh.at[pl.ds(s * RA + i * CH, CH)])

    pltpu.sync_copy(rows_a.at[pl.ds(0, RA - (RA // CH) * CH)],
                    acc_sh.at[pl.ds(s * RA + (RA // CH) * CH,
                                    RA - (RA // CH) * CH)])

    @pl.when(s == NS - 1)
    def _():
      pltpu.sync_copy(rows_a.at[pl.ds(0, RTAIL)],
                      acc_sh.at[pl.ds(NS * RA, RTAIL)])

    plsc.subcore_barrier()

    pltpu.sync_copy(src_hbm.at[pl.ds(wid * EPT, EPT)], src_v)
    pltpu.sync_copy(dst_hbm.at[wid], dst_v)

    def _start(j, buf, dsem):
      pltpu.async_copy(y_hbm.at[src_v.at[pl.ds(j * CH, CH)]], buf, dsem)

    def _drain(buf, dsem):
      pltpu.make_async_copy(y_hbm.at[src_v.at[pl.ds(0, CH)]], buf, dsem).wait()

    _start(0, rows_a, sem_a)

    @pl.loop(0, (NCH - 1) // 2)
    def _(k):
      j = 2 * k
      _start(j + 1, rows_b, sem_b)
      _drain(rows_a, sem_a)
      pltpu.sync_copy(rows_a, acc_sh.at[dst_v.at[j]], add=True)
      _start(j + 2, rows_a, sem_a)
      _drain(rows_b, sem_b)
      pltpu.sync_copy(rows_b, acc_sh.at[dst_v.at[j + 1]], add=True)

    _drain(rows_a, sem_a)
    pltpu.sync_copy(rows_a, acc_sh.at[dst_v.at[NCH - 1]], add=True)

    plsc.subcore_barrier()

    pltpu.sync_copy(acc_sh.at[pl.ds(s * RA, RA)],
                    out_hbm.at[c, pl.ds(s * RA, RA)])

    @pl.when(s == NS - 1)
    def _():
      pltpu.sync_copy(acc_sh.at[pl.ds(NS * RA, RTAIL)],
                      out_hbm.at[c, pl.ds(NS * RA, RTAIL)])

  return agg


_agg1 = _make_agg(W1)
_agg2 = _make_agg(W2)


def _lin1_body(x_ref, wl_ref, wr_ref, y_ref, xr_ref):
  x = x_ref[...]
  y = lax.dot_general(x, wl_ref[...], (((1,), (1,)), ((), ())),
                      preferred_element_type=jnp.float32)
  ones = jnp.ones((x.shape[0], 1), jnp.float32)
  zeros = jnp.zeros((x.shape[0], W1 - HID - 1), jnp.float32)
  y_ref[...] = jnp.concatenate([y, ones, zeros], axis=1)
  xr_ref[...] = lax.dot_general(x, wr_ref[...], (((1,), (1,)), ((), ())),
                                preferred_element_type=jnp.float32)


def _mid_body(p_ref, xr_ref, bl1_ref, wl2_ref, wr2_ref, y2_ref, hr2_ref,
              deg_ref):
  p = p_ref[0] + p_ref[1]
  deg = jnp.maximum(p[:, HID:HID + 1], 1.0)
  h = jnp.maximum(p[:, :HID] / deg + bl1_ref[...] + xr_ref[...], 0.0)
  y2 = lax.dot_general(h, wl2_ref[...], (((1,), (1,)), ((), ())),
                       preferred_element_type=jnp.float32)
  y2_ref[...] = jnp.concatenate(
      [y2, jnp.zeros((y2.shape[0], W2 - HID), jnp.float32)], axis=1)
  hr2_ref[...] = lax.dot_general(h, wr2_ref[...], (((1,), (1,)), ((), ())),
                                 preferred_element_type=jnp.float32)
  deg_ref[...] = deg


def _out_body(p_ref, hr2_ref, bl2_ref, deg_ref, wout_ref, bout_ref, o_ref):
  agg = p_ref[0][:, :HID] + p_ref[1][:, :HID]
  h = jnp.maximum(agg / deg_ref[...] + bl2_ref[...] + hr2_ref[...], 0.0)
  o_ref[...] = lax.dot_general(h, wout_ref[...], (((1,), (1,)), ((), ())),
                               preferred_element_type=jnp.float32) + bout_ref[...]


def kernel(x, edge_index, Wl1, bl1, Wr1, Wl2, bl2, Wr2, Wout, bout):
  src = edge_index[0]
  dst = edge_index[1].reshape(NW, NCH, CH)

  y1p, xr1 = pl.pallas_call(
      _lin1_body,
      out_shape=(jax.ShapeDtypeStruct((N, W1), jnp.float32),
                 jax.ShapeDtypeStruct((N, HID), jnp.float32)),
  )(x, Wl1, Wr1)

  p1 = _agg1(y1p, src, dst)

  y2, hr2, deg = pl.pallas_call(
      _mid_body,
      out_shape=(jax.ShapeDtypeStruct((N, W2), jnp.float32),
                 jax.ShapeDtypeStruct((N, HID), jnp.float32),
                 jax.ShapeDtypeStruct((N, 1), jnp.float32)),
  )(p1, xr1, bl1.reshape(1, HID), Wl2, Wr2)

  p2 = _agg2(y2, src, dst)

  out = pl.pallas_call(
      _out_body,
      out_shape=jax.ShapeDtypeStruct((N, OUT), jnp.float32),
  )(p2, hr2, bl2.reshape(1, HID), deg, Wout, bout.reshape(1, OUT))

  return out


# speedup vs baseline: 1.0177x; 1.0177x over previous
import functools

import jax
import jax.numpy as jnp
from jax import lax
from jax.experimental import pallas as pl
from jax.experimental.pallas import tpu as pltpu
from jax.experimental.pallas import tpu_sc as plsc

N = 10000
E = 320000
D_IN = 128
HID = 64
OUT = 112

NC = 2
NS = 16
NW = NC * NS
EPT = E // NW
CH = 80
NCH = EPT // CH
RA = 624
RTAIL = N - NS * RA
W1 = 128
W2 = 128


def _make_agg(W):
  mesh = plsc.VectorSubcoreMesh(core_axis_name="c", subcore_axis_name="s")

  @functools.partial(
      pl.kernel,
      out_type=jax.ShapeDtypeStruct((NC, N, W), jnp.float32),
      mesh=mesh,
      scratch_types=[
          pltpu.VMEM((EPT,), jnp.int32),
          pltpu.VMEM((NCH, CH), jnp.int32),
          pltpu.VMEM((CH, W), jnp.float32),
          pltpu.VMEM((CH, W), jnp.float32),
          pltpu.VMEM_SHARED((N, W), jnp.float32),
          pltpu.SemaphoreType.DMA,
          pltpu.SemaphoreType.DMA,
      ],
  )
  def agg(y_hbm, src_hbm, dst_hbm, out_hbm, src_v, dst_v, rows_a, rows_b,
          acc_sh, sem_a, sem_b):
    c = lax.axis_index("c")
    s = lax.axis_index("s")
    wid = s * NC + c

    pltpu.async_copy(src_hbm.at[pl.ds(wid * EPT, EPT)], src_v, sem_a)
    pltpu.async_copy(dst_hbm.at[wid], dst_v, sem_b)

    @pl.loop(0, CH)
    def _(i):
      @pl.loop(0, W // 16)
      def _(k):
        rows_b[i, pl.ds(k * 16, 16)] = jnp.zeros((16,), jnp.float32)

    pltpu.make_async_copy(src_hbm.at[pl.ds(0, EPT)], src_v, sem_a).wait()
    pltpu.make_async_copy(dst_hbm.at[0], dst_v, sem_b).wait()

    def _start(j, buf, dsem):
      pltpu.async_copy(y_hbm.at[src_v.at[pl.ds(j * CH, CH)]], buf, dsem)

    def _drain(buf, dsem):
      pltpu.make_async_copy(y_hbm.at[src_v.at[pl.ds(0, CH)]], buf, dsem).wait()

    _start(0, rows_a, sem_a)

    @pl.loop(0, RA // CH)
    def _(i):
      pltpu.sync_copy(rows_b, acc_sh.at[pl.ds(s * RA + i * CH, CH)])

    pltpu.sync_copy(rows_b.at[pl.ds(0, RA - (RA // CH) * CH)],
                    acc_sh.at[pl.ds(s * RA + (RA // CH) * CH,
                                    RA - (RA // CH) * CH)])

    @pl.when(s == NS - 1)
    def _():
      pltpu.sync_copy(rows_b.at[pl.ds(0, RTAIL)],
                      acc_sh.at[pl.ds(NS * RA, RTAIL)])

    plsc.subcore_barrier()

    @pl.loop(0, (NCH - 1) // 2)
    def _(k):
      j = 2 * k
      _start(j + 1, rows_b, sem_b)
      _drain(rows_a, sem_a)
      pltpu.sync_copy(rows_a, acc_sh.at[dst_v.at[j]], add=True)
      _start(j + 2, rows_a, sem_a)
      _drain(rows_b, sem_b)
      pltpu.sync_copy(rows_b, acc_sh.at[dst_v.at[j + 1]], add=True)

    _drain(rows_a, sem_a)
    pltpu.sync_copy(rows_a, acc_sh.at[dst_v.at[NCH - 1]], add=True)

    plsc.subcore_barrier()

    pltpu.sync_copy(acc_sh.at[pl.ds(s * RA, RA)],
                    out_hbm.at[c, pl.ds(s * RA, RA)])

    @pl.when(s == NS - 1)
    def _():
      pltpu.sync_copy(acc_sh.at[pl.ds(NS * RA, RTAIL)],
                      out_hbm.at[c, pl.ds(NS * RA, RTAIL)])

  return agg


_agg1 = _make_agg(W1)
_agg2 = _make_agg(W2)


def _lin1_body(x_ref, wl_ref, wr_ref, y_ref, xr_ref):
  x = x_ref[...]
  y = lax.dot_general(x, wl_ref[...], (((1,), (1,)), ((), ())),
                      preferred_element_type=jnp.float32)
  ones = jnp.ones((x.shape[0], 1), jnp.float32)
  zeros = jnp.zeros((x.shape[0], W1 - HID - 1), jnp.float32)
  y_ref[...] = jnp.concatenate([y, ones, zeros], axis=1)
  xr_ref[...] = lax.dot_general(x, wr_ref[...], (((1,), (1,)), ((), ())),
                                preferred_element_type=jnp.float32)


def _mid_body(p_ref, xr_ref, bl1_ref, wl2_ref, wr2_ref, y2_ref, hr2_ref,
              deg_ref):
  p = p_ref[0] + p_ref[1]
  deg = jnp.maximum(p[:, HID:HID + 1], 1.0)
  h = jnp.maximum(p[:, :HID] / deg + bl1_ref[...] + xr_ref[...], 0.0)
  y2 = lax.dot_general(h, wl2_ref[...], (((1,), (1,)), ((), ())),
                       preferred_element_type=jnp.float32)
  y2_ref[...] = jnp.concatenate(
      [y2, jnp.zeros((y2.shape[0], W2 - HID), jnp.float32)], axis=1)
  hr2_ref[...] = lax.dot_general(h, wr2_ref[...], (((1,), (1,)), ((), ())),
                                 preferred_element_type=jnp.float32)
  deg_ref[...] = deg


def _out_body(p_ref, hr2_ref, bl2_ref, deg_ref, wout_ref, bout_ref, o_ref):
  agg = p_ref[0][:, :HID] + p_ref[1][:, :HID]
  h = jnp.maximum(agg / deg_ref[...] + bl2_ref[...] + hr2_ref[...], 0.0)
  o_ref[...] = lax.dot_general(h, wout_ref[...], (((1,), (1,)), ((), ())),
                               preferred_element_type=jnp.float32) + bout_ref[...]


def kernel(x, edge_index, Wl1, bl1, Wr1, Wl2, bl2, Wr2, Wout, bout):
  src = edge_index[0]
  dst = edge_index[1].reshape(NW, NCH, CH)

  y1p, xr1 = pl.pallas_call(
      _lin1_body,
      out_shape=(jax.ShapeDtypeStruct((N, W1), jnp.float32),
                 jax.ShapeDtypeStruct((N, HID), jnp.float32)),
  )(x, Wl1, Wr1)

  p1 = _agg1(y1p, src, dst)

  y2, hr2, deg = pl.pallas_call(
      _mid_body,
      out_shape=(jax.ShapeDtypeStruct((N, W2), jnp.float32),
                 jax.ShapeDtypeStruct((N, HID), jnp.float32),
                 jax.ShapeDtypeStruct((N, 1), jnp.float32)),
  )(p1, xr1, bl1.reshape(1, HID), Wl2, Wr2)

  p2 = _agg2(y2, src, dst)

  out = pl.pallas_call(
      _out_body,
      out_shape=jax.ShapeDtypeStruct((N, OUT), jnp.float32),
  )(p2, hr2, bl2.reshape(1, HID), deg, Wout, bout.reshape(1, OUT))

  return out
